# baseline (speedup 1.0000x reference)
import functools
import jax
import jax.numpy as jnp
from jax import lax
from jax.experimental import pallas as pl
from jax.experimental.pallas import tpu as pltpu
from jax.experimental.pallas import tpu_sc as plsc

N, E, D, DE, H, L = 10000, 320000, 128, 16, 64, 3

_INTERPRET = False


def _node_body(x_ref, w1_ref, b1_ref, w2_ref, b2_ref, lng_ref, lnb_ref,
               encw_ref, encb_ref, bng_ref, bnb_ref, h_ref):
    t = jnp.maximum(x_ref[...] @ w1_ref[...] + b1_ref[...], 0.0)
    xp = t @ w2_ref[...] + b2_ref[...]
    m = jnp.mean(xp, axis=1, keepdims=True)
    v = jnp.mean((xp - m) ** 2, axis=1, keepdims=True)
    xp = (xp - m) / jnp.sqrt(v + 1e-5) * lng_ref[...] + lnb_ref[...]
    y = xp @ encw_ref[...] + encb_ref[...]
    mu = jnp.mean(y, axis=0, keepdims=True)
    var = jnp.mean((y - mu) ** 2, axis=0, keepdims=True)
    h_ref[...] = jnp.maximum(
        (y - mu) / jnp.sqrt(var + 1e-5) * bng_ref[...] + bnb_ref[...], 0.0)


def _node_dense(x, w1, b1, w2, b2, lng, lnb, encw, encb, bng, bnb):
    return pl.pallas_call(
        _node_body,
        out_shape=jax.ShapeDtypeStruct((N, H), jnp.float32),
        interpret=_INTERPRET,
    )(x, w1, b1.reshape(1, -1), w2, b2.reshape(1, -1), lng.reshape(1, -1),
      lnb.reshape(1, -1), encw, encb.reshape(1, -1), bng.reshape(1, -1),
      bnb.reshape(1, -1))


_BE = 6400


def _edge_body(eattr_ref, w1_ref, b1_ref, w2_ref, b2_ref, lng_ref, lnb_ref,
               lew_ref, ate_ref, ea_ref, ae_ref, me_ref):
    t = jnp.maximum(eattr_ref[...] @ w1_ref[...] + b1_ref[...], 0.0)
    ea = t @ w2_ref[...] + b2_ref[...]
    m = jnp.mean(ea, axis=1, keepdims=True)
    v = jnp.mean((ea - m) ** 2, axis=1, keepdims=True)
    ea = (ea - m) / jnp.sqrt(v + 1e-5) * lng_ref[...] + lnb_ref[...]
    ea_ref[...] = ea
    lew = lew_ref[...]
    ate = ate_ref[...]
    we = jnp.sum(lew * ate[:, None, :], axis=2)
    ae = ea @ we.T
    ae_ref[...] = ae
    bmax = jnp.max(ae, axis=0, keepdims=True)
    @pl.when(pl.program_id(0) == 0)
    def _():
        me_ref[...] = bmax
    @pl.when(pl.program_id(0) != 0)
    def _():
        me_ref[...] = jnp.maximum(me_ref[...], bmax)


def _edge_dense(edge_attr, w1, b1, w2, b2, lng, lnb, lew, ate):
    nb = E // _BE
    return pl.pallas_call(
        _edge_body,
        grid=(nb,),
        in_specs=[
            pl.BlockSpec((_BE, DE), lambda i: (i, 0)),
            pl.BlockSpec((DE, 2 * DE), lambda i: (0, 0)),
            pl.BlockSpec((1, 2 * DE), lambda i: (0, 0)),
            pl.BlockSpec((2 * DE, DE), lambda i: (0, 0)),
            pl.BlockSpec((1, DE), lambda i: (0, 0)),
            pl.BlockSpec((1, DE), lambda i: (0, 0)),
            pl.BlockSpec((1, DE), lambda i: (0, 0)),
            pl.BlockSpec((L, DE, H), lambda i: (0, 0, 0)),
            pl.BlockSpec((L, H), lambda i: (0, 0)),
        ],
        out_specs=[
            pl.BlockSpec((_BE, DE), lambda i: (i, 0)),
            pl.BlockSpec((_BE, L), lambda i: (i, 0)),
            pl.BlockSpec((1, L), lambda i: (0, 0)),
        ],
        out_shape=[
            jax.ShapeDtypeStruct((E, DE), jnp.float32),
            jax.ShapeDtypeStruct((E, L), jnp.float32),
            jax.ShapeDtypeStruct((1, L), jnp.float32),
        ],
        interpret=_INTERPRET,
    )(edge_attr, w1, b1.reshape(1, -1), w2, b2.reshape(1, -1),
      lng.reshape(1, -1), lnb.reshape(1, -1), lew, ate)


def _front_body(h_ref, linw_ref, ats_ref, atd_ref, me_ref,
                hw_ref, as_ref, ad_ref, m_ref):
    hw = h_ref[...] @ linw_ref[...]
    hw_ref[...] = hw
    a_s = hw @ ats_ref[...]
    a_d = hw @ atd_ref[...]
    as_ref[...] = a_s
    ad_ref[...] = a_d
    m = jnp.maximum(jnp.max(a_s) + jnp.max(a_d)
                    + jnp.maximum(me_ref[0, 0], 0.0), 0.0)
    m_ref[...] = jnp.full((1, 128), m, jnp.float32)


def _layer_front(h, linw, ats, atd, me_l):
    return pl.pallas_call(
        _front_body,
        out_shape=[
            jax.ShapeDtypeStruct((N, H), jnp.float32),
            jax.ShapeDtypeStruct((N, 1), jnp.float32),
            jax.ShapeDtypeStruct((N, 1), jnp.float32),
            jax.ShapeDtypeStruct((1, 128), jnp.float32),
        ],
        interpret=_INTERPRET,
    )(h, linw, ats.reshape(-1, 1), atd.reshape(-1, 1), me_l.reshape(1, 1))


def _epi_body(u_ref, dn_ref, as_ref, ad_ref, s_ref, we_ref, deg_ref,
              hw_ref, hin_ref, m_ref, bias_ref, g_ref, b_ref, h_ref):
    u = jnp.sum(u_ref[...], axis=0)
    denom = jnp.sum(dn_ref[...], axis=0)
    ael = (s_ref[...] @ we_ref[...]) / jnp.maximum(deg_ref[...], 1.0)
    zs = as_ref[...] + ad_ref[...] + ael
    zs = jnp.maximum(zs, 0.2 * zs)
    exs = jnp.exp(zs - m_ref[0, 0])
    hw = hw_ref[...]
    out = (u + exs * hw) / (denom + exs + 1e-16) + bias_ref[...]
    mu = jnp.mean(out, axis=0, keepdims=True)
    var = jnp.mean((out - mu) ** 2, axis=0, keepdims=True)
    hn = jnp.maximum((out - mu) / jnp.sqrt(var + 1e-5) * g_ref[...]
                     + b_ref[...], 0.0)
    h_ref[...] = hin_ref[...] + hn


def _epilogue(u, dn, a_s, a_d, S, we_l, deg, hw, h_in, m, bias, g, b):
    return pl.pallas_call(
        _epi_body,
        out_shape=jax.ShapeDtypeStruct((N, H), jnp.float32),
        interpret=_INTERPRET,
    )(u, dn, a_s, a_d, S, we_l.reshape(-1, 1), deg, hw, h_in,
      m.reshape(1, -1), bias.reshape(1, -1), g.reshape(1, -1),
      b.reshape(1, -1))


def _final_body(h_ref, ow_ref, ob_ref, o_ref):
    g = jnp.mean(h_ref[...], axis=0, keepdims=True)
    o_ref[...] = jnp.maximum(g @ ow_ref[...] + ob_ref[...], 0.0)


def _final(h, out_w, out_b):
    return pl.pallas_call(
        _final_body,
        out_shape=jax.ShapeDtypeStruct((1, H), jnp.float32),
        interpret=_INTERPRET,
    )(h, out_w, out_b.reshape(1, -1))


def kernel(x, edge_index, edge_attr, npp_w1, npp_b1, npp_w2, npp_b2,
           npp_ln_g, npp_ln_b, epp_w1, epp_b1, epp_w2, epp_b2, epp_ln_g,
           epp_ln_b, enc_w, enc_b, enc_bn_g, enc_bn_b, gat_lin_w,
           gat_att_src, gat_att_dst, gat_lin_edge_w, gat_att_edge,
           gat_bias, bn_g, bn_b, out_w, out_b):
    src, dst = edge_index[0], edge_index[1]
    h = _node_dense(x, npp_w1, npp_b1, npp_w2, npp_b2, npp_ln_g, npp_ln_b,
                    enc_w, enc_b, enc_bn_g, enc_bn_b)
    ea, aeE, me = _edge_dense(edge_attr, epp_w1, epp_b1, epp_w2, epp_b2,
                              epp_ln_g, epp_ln_b, gat_lin_edge_w,
                              gat_att_edge)
    We = jnp.sum(gat_lin_edge_w * gat_att_edge[:, None, :], axis=2)

    deg = jax.ops.segment_sum(jnp.ones((E,), jnp.float32), dst,
                              num_segments=N)
    S = jax.ops.segment_sum(ea, dst, num_segments=N)

    for l in range(L):
        hw, a_s, a_d, m = _layer_front(h, gat_lin_w[l], gat_att_src[l],
                                       gat_att_dst[l], me[0, l])
        ae = aeE[:, l]
        z = a_s[src, 0] + a_d[dst, 0] + ae
        ex = jnp.exp(jnp.maximum(z, 0.2 * z) - m[0, 0])
        denom = jax.ops.segment_sum(ex, dst, num_segments=N)
        u = jax.ops.segment_sum(ex[:, None] * hw[src], dst, num_segments=N)
        h = _epilogue(u[None], denom[None, :, None], a_s, a_d, S, We[l],
                      deg[:, None], hw, h, m, gat_bias[l], bn_g[l], bn_b[l])
    return _final(h, out_w, out_b)


# speedup vs baseline: 21.0425x; 21.0425x over previous
import functools
import jax
import jax.numpy as jnp
from jax import lax
from jax.experimental import pallas as pl
from jax.experimental.pallas import tpu as pltpu
from jax.experimental.pallas import tpu_sc as plsc

N, E, D, DE, H, L = 10000, 320000, 128, 16, 64, 3

_INTERPRET = False


def _node_body(x_ref, w1_ref, b1_ref, w2_ref, b2_ref, lng_ref, lnb_ref,
               encw_ref, encb_ref, bng_ref, bnb_ref, h_ref):
    t = jnp.maximum(x_ref[...] @ w1_ref[...] + b1_ref[...], 0.0)
    xp = t @ w2_ref[...] + b2_ref[...]
    m = jnp.mean(xp, axis=1, keepdims=True)
    v = jnp.mean((xp - m) ** 2, axis=1, keepdims=True)
    xp = (xp - m) / jnp.sqrt(v + 1e-5) * lng_ref[...] + lnb_ref[...]
    y = xp @ encw_ref[...] + encb_ref[...]
    mu = jnp.mean(y, axis=0, keepdims=True)
    var = jnp.mean((y - mu) ** 2, axis=0, keepdims=True)
    h_ref[...] = jnp.maximum(
        (y - mu) / jnp.sqrt(var + 1e-5) * bng_ref[...] + bnb_ref[...], 0.0)


def _node_dense(x, w1, b1, w2, b2, lng, lnb, encw, encb, bng, bnb):
    return pl.pallas_call(
        _node_body,
        out_shape=jax.ShapeDtypeStruct((N, H), jnp.float32),
        interpret=_INTERPRET,
    )(x, w1, b1.reshape(1, -1), w2, b2.reshape(1, -1), lng.reshape(1, -1),
      lnb.reshape(1, -1), encw, encb.reshape(1, -1), bng.reshape(1, -1),
      bnb.reshape(1, -1))


_BE = 6400


def _edge_body(eattr_ref, w1_ref, b1_ref, w2_ref, b2_ref, lng_ref, lnb_ref,
               lew_ref, ate_ref, ea32_ref, ae_ref, me_ref):
    t = jnp.maximum(eattr_ref[...] @ w1_ref[...] + b1_ref[...], 0.0)
    ea = t @ w2_ref[...] + b2_ref[...]
    m = jnp.mean(ea, axis=1, keepdims=True)
    v = jnp.mean((ea - m) ** 2, axis=1, keepdims=True)
    ea = (ea - m) / jnp.sqrt(v + 1e-5) * lng_ref[...] + lnb_ref[...]
    ea32_ref[...] = jnp.concatenate(
        [ea, jnp.full((ea.shape[0], 1), 1.0, jnp.float32),
         jnp.zeros((ea.shape[0], 32 - DE - 1), jnp.float32)], axis=1)
    lew = lew_ref[...]
    ate = ate_ref[...]
    we = jnp.sum(lew * ate[:, None, :], axis=2)
    ae = ea @ we.T
    ae_ref[...] = ae.T
    bmax = jnp.max(ae, axis=0, keepdims=True)
    @pl.when(pl.program_id(0) == 0)
    def _():
        me_ref[...] = bmax
    @pl.when(pl.program_id(0) != 0)
    def _():
        me_ref[...] = jnp.maximum(me_ref[...], bmax)


def _edge_dense(edge_attr, w1, b1, w2, b2, lng, lnb, lew, ate):
    nb = E // _BE
    return pl.pallas_call(
        _edge_body,
        grid=(nb,),
        in_specs=[
            pl.BlockSpec((_BE, DE), lambda i: (i, 0)),
            pl.BlockSpec((DE, 2 * DE), lambda i: (0, 0)),
            pl.BlockSpec((1, 2 * DE), lambda i: (0, 0)),
            pl.BlockSpec((2 * DE, DE), lambda i: (0, 0)),
            pl.BlockSpec((1, DE), lambda i: (0, 0)),
            pl.BlockSpec((1, DE), lambda i: (0, 0)),
            pl.BlockSpec((1, DE), lambda i: (0, 0)),
            pl.BlockSpec((L, DE, H), lambda i: (0, 0, 0)),
            pl.BlockSpec((L, H), lambda i: (0, 0)),
        ],
        out_specs=[
            pl.BlockSpec((_BE, 32), lambda i: (i, 0)),
            pl.BlockSpec((L, _BE), lambda i: (0, i)),
            pl.BlockSpec((1, L), lambda i: (0, 0)),
        ],
        out_shape=[
            jax.ShapeDtypeStruct((E, 32), jnp.float32),
            jax.ShapeDtypeStruct((L, E), jnp.float32),
            jax.ShapeDtypeStruct((1, L), jnp.float32),
        ],
        interpret=_INTERPRET,
    )(edge_attr, w1, b1.reshape(1, -1), w2, b2.reshape(1, -1),
      lng.reshape(1, -1), lnb.reshape(1, -1), lew, ate)


def _front_body(h_ref, linw_ref, ats_ref, atd_ref, me_ref,
                hw_ref, as_ref, ad_ref, m_ref):
    hw = h_ref[...] @ linw_ref[...]
    hw_ref[...] = jnp.concatenate(
        [hw, jnp.full((hw.shape[0], 1), 1.0, jnp.float32),
         jnp.zeros((hw.shape[0], 80 - H - 1), jnp.float32)], axis=1)
    a_s = hw @ ats_ref[...]
    a_d = hw @ atd_ref[...]
    as_ref[...] = a_s
    ad_ref[...] = a_d
    m = jnp.maximum(jnp.max(a_s) + jnp.max(a_d)
                    + jnp.maximum(me_ref[0, 0], 0.0), 0.0)
    m_ref[...] = jnp.full((1, 128), m, jnp.float32)


def _layer_front(h, linw, ats, atd, me_l):
    return pl.pallas_call(
        _front_body,
        out_shape=[
            jax.ShapeDtypeStruct((N, 80), jnp.float32),
            jax.ShapeDtypeStruct((N, 1), jnp.float32),
            jax.ShapeDtypeStruct((N, 1), jnp.float32),
            jax.ShapeDtypeStruct((1, 128), jnp.float32),
        ],
        interpret=_INTERPRET,
    )(h, linw, ats.reshape(-1, 1), atd.reshape(-1, 1), me_l.reshape(1, 1))


def _epilogue(u2, a_s, a_d, Sp, we_l, hw80, h_in, m, bias, g, b):
    def body(u_ref, as_ref, ad_ref, s_ref, we_ref, hw_ref, hin_ref,
             m_ref, bias_ref, g_ref, b_ref, h_ref):
        up = jnp.sum(u_ref[...], axis=0)
        u = up[:, :H]
        denom = up[:, H:H + 1]
        sp = jnp.sum(s_ref[...], axis=0)
        deg = sp[:, DE:DE + 1]
        ael = (sp[:, :DE] @ we_ref[...]) / jnp.maximum(deg, 1.0)
        zs = as_ref[...] + ad_ref[...] + ael
        zs = jnp.maximum(zs, 0.2 * zs)
        exs = jnp.exp(zs - m_ref[0, 0])
        hw = hw_ref[...][:, :H]
        out = (u + exs * hw) / (denom + exs + 1e-16) + bias_ref[...]
        mu = jnp.mean(out, axis=0, keepdims=True)
        var = jnp.mean((out - mu) ** 2, axis=0, keepdims=True)
        hn = jnp.maximum((out - mu) / jnp.sqrt(var + 1e-5) * g_ref[...]
                         + b_ref[...], 0.0)
        h_ref[...] = hin_ref[...] + hn
    return pl.pallas_call(
        body,
        out_shape=jax.ShapeDtypeStruct((N, H), jnp.float32),
        interpret=_INTERPRET,
    )(u2, a_s, a_d, Sp, we_l.reshape(-1, 1), hw80, h_in,
      m.reshape(1, -1), bias.reshape(1, -1), g.reshape(1, -1),
      b.reshape(1, -1))


def _final_body(h_ref, ow_ref, ob_ref, o_ref):
    g = jnp.mean(h_ref[...], axis=0, keepdims=True)
    o_ref[...] = jnp.maximum(g @ ow_ref[...] + ob_ref[...], 0.0)


def _final(h, out_w, out_b):
    return pl.pallas_call(
        _final_body,
        out_shape=jax.ShapeDtypeStruct((1, H), jnp.float32),
        interpret=_INTERPRET,
    )(h, out_w, out_b.reshape(1, -1))


_NC, _NS = 2, 16
_NW = _NC * _NS
_EW = E // _NW
_C = 128
_NF = _EW // _C
_TAIL = _EW - _NF * _C
_RPT = N // _NS
_SC_MESH = plsc.VectorSubcoreMesh(core_axis_name="c", subcore_axis_name="s")


def _zero_rows(ref, lo, nrows, width16):
    def body(i, _):
        for j in range(width16):
            ref[i, pl.ds(j * 16, 16)] = jnp.zeros((16,), jnp.float32)
        return 0
    lax.fori_loop(lo, lo + nrows, body, 0)


def _acc_blocks(sid):
    base = sid * _RPT
    return [(base, _C), (base + _C, _C), (base + 2 * _C, _C),
            (base + 3 * _C, _C), (base + 4 * _C, _RPT - 4 * _C)]


def _seg_body(dst2_hbm, ea32_hbm, sp_hbm, chunk_v, idx_v, s_sh):
    cid = lax.axis_index("c")
    sid = lax.axis_index("s")
    wid = sid * _NC + cid
    _zero_rows(chunk_v, 0, _C, 2)
    for off, nr in _acc_blocks(sid):
        pltpu.sync_copy(chunk_v.at[pl.ds(0, nr)], s_sh.at[pl.ds(off, nr)])
    plsc.subcore_barrier()

    def body(c, _):
        pltpu.sync_copy(dst2_hbm.at[wid, pl.ds(c * _C, _C)], idx_v)
        pltpu.sync_copy(ea32_hbm.at[pl.ds(wid * _EW + c * _C, _C)], chunk_v)
        pltpu.sync_copy(chunk_v, s_sh.at[idx_v], add=True)
        return 0
    lax.fori_loop(0, _NF, body, 0)
    _zero_rows(chunk_v, 0, _C, 2)
    pltpu.sync_copy(dst2_hbm.at[wid, pl.ds(_NF * _C, _C)], idx_v)
    pltpu.sync_copy(ea32_hbm.at[pl.ds(wid * _EW + _NF * _C, _TAIL)],
                    chunk_v.at[pl.ds(0, _TAIL)])
    pltpu.sync_copy(chunk_v, s_sh.at[idx_v], add=True)
    plsc.subcore_barrier()
    for off, nr in _acc_blocks(sid):
        pltpu.sync_copy(s_sh.at[pl.ds(off, nr)],
                        sp_hbm.at[cid, pl.ds(off, nr)])


def _seg_ea(dst2, ea32):
    return pl.kernel(
        _seg_body,
        out_type=jax.ShapeDtypeStruct((_NC, N, 32), jnp.float32),
        mesh=_SC_MESH,
        compiler_params=pltpu.CompilerParams(use_tc_tiling_on_sc=False, needs_layout_passes=False),
        scratch_types=[
            pltpu.VMEM((_C, 32), jnp.float32),
            pltpu.VMEM((_C,), jnp.int32),
            pltpu.VMEM_SHARED((N, 32), jnp.float32),
        ],
    )(dst2, ea32)


_EP = _EW + 112


def _gat_body(src2_hbm, dst2_hbm, ae2_hbm, as_hbm, ad_hbm, m_hbm, hw_hbm,
              up_hbm, asv, adv, srcv, dstv, aev, m16, exv, src128, dst128,
              rows, u_sh):
    cid = lax.axis_index("c")
    sid = lax.axis_index("s")
    wid = sid * _NC + cid
    pltpu.sync_copy(as_hbm, asv)
    pltpu.sync_copy(ad_hbm, adv)
    pltpu.sync_copy(m_hbm.at[0, pl.ds(0, 16)], m16)
    pltpu.sync_copy(src2_hbm.at[wid], srcv)
    pltpu.sync_copy(dst2_hbm.at[wid], dstv)
    pltpu.sync_copy(ae2_hbm.at[wid], aev)
    _zero_rows(rows, 0, _C, 5)
    for off, nr in _acc_blocks(sid):
        pltpu.sync_copy(rows.at[pl.ds(0, nr)], u_sh.at[pl.ds(off, nr)])
    plsc.subcore_barrier()
    mvec = m16[...]

    def chunk(c, _):
        off = c * _C

        def scal(i, _):
            sl = pl.ds(off + i * 16, 16)
            z = (plsc.load_gather(asv, [srcv[sl]])
                 + plsc.load_gather(adv, [dstv[sl]]) + aev[sl])
            z = jnp.maximum(z, 0.2 * z)
            exv[pl.ds(i * 16, 16)] = jnp.exp(z - mvec)
            return 0
        lax.fori_loop(0, 8, scal, 0)
        pltpu.sync_copy(src2_hbm.at[wid, pl.ds(off, _C)], src128)
        pltpu.sync_copy(dst2_hbm.at[wid, pl.ds(off, _C)], dst128)
        pltpu.sync_copy(hw_hbm.at[src128], rows)

        def scale(e, _):
            bc = plsc.load_gather(exv, [jnp.full((16,), e, jnp.int32)])
            for j in range(5):
                sl = pl.ds(j * 16, 16)
                rows[e, sl] = rows[e, sl] * bc
            return 0
        lax.fori_loop(0, _C, scale, 0)
        pltpu.sync_copy(rows, u_sh.at[dst128], add=True)
        return 0
    lax.fori_loop(0, _NF + 1, chunk, 0)
    plsc.subcore_barrier()
    for off, nr in _acc_blocks(sid):
        pltpu.sync_copy(u_sh.at[pl.ds(off, nr)],
                        up_hbm.at[cid, pl.ds(off, nr)])


def _gat_edges(src2, dst2, ae2, a_s, a_d, m, hw80):
    return pl.kernel(
        _gat_body,
        out_type=jax.ShapeDtypeStruct((_NC, N, 80), jnp.float32),
        mesh=_SC_MESH,
        compiler_params=pltpu.CompilerParams(use_tc_tiling_on_sc=False, needs_layout_passes=False),
        scratch_types=[
            pltpu.VMEM((N,), jnp.float32),
            pltpu.VMEM((N,), jnp.float32),
            pltpu.VMEM((_EP,), jnp.int32),
            pltpu.VMEM((_EP,), jnp.int32),
            pltpu.VMEM((_EP,), jnp.float32),
            pltpu.VMEM((16,), jnp.float32),
            pltpu.VMEM((_C,), jnp.float32),
            pltpu.VMEM((_C,), jnp.int32),
            pltpu.VMEM((_C,), jnp.int32),
            pltpu.VMEM((_C, 80), jnp.float32),
            pltpu.VMEM_SHARED((N, 80), jnp.float32),
        ],
    )(src2, dst2, ae2, a_s, a_d, m, hw80)


def kernel(x, edge_index, edge_attr, npp_w1, npp_b1, npp_w2, npp_b2,
           npp_ln_g, npp_ln_b, epp_w1, epp_b1, epp_w2, epp_b2, epp_ln_g,
           epp_ln_b, enc_w, enc_b, enc_bn_g, enc_bn_b, gat_lin_w,
           gat_att_src, gat_att_dst, gat_lin_edge_w, gat_att_edge,
           gat_bias, bn_g, bn_b, out_w, out_b):
    src, dst = edge_index[0], edge_index[1]
    h = _node_dense(x, npp_w1, npp_b1, npp_w2, npp_b2, npp_ln_g, npp_ln_b,
                    enc_w, enc_b, enc_bn_g, enc_bn_b)
    ea32, aeT, me = _edge_dense(edge_attr, epp_w1, epp_b1, epp_w2, epp_b2,
                                epp_ln_g, epp_ln_b, gat_lin_edge_w,
                                gat_att_edge)
    We = jnp.sum(gat_lin_edge_w * gat_att_edge[:, None, :], axis=2)
    src2 = jnp.concatenate([src.reshape(_NW, _EW),
                            jnp.zeros((_NW, _EP - _EW), jnp.int32)], axis=1)
    dst2 = jnp.concatenate([dst.reshape(_NW, _EW),
                            jnp.zeros((_NW, _EP - _EW), jnp.int32)], axis=1)
    ae3 = jnp.concatenate(
        [aeT.reshape(L, _NW, _EW),
         jnp.full((L, _NW, _EP - _EW), -1e30, jnp.float32)], axis=2)
    Sp = _seg_ea(dst2, ea32)

    for l in range(L):
        hw80, a_s, a_d, m = _layer_front(h, gat_lin_w[l], gat_att_src[l],
                                         gat_att_dst[l], me[0, l])
        u2 = _gat_edges(src2, dst2, ae3[l], a_s.reshape(N), a_d.reshape(N),
                        m, hw80)
        h = _epilogue(u2, a_s, a_d, Sp, We[l], hw80, h, m,
                      gat_bias[l], bn_g[l], bn_b[l])
    return _final(h, out_w, out_b)


# speedup vs baseline: 23.0904x; 1.0973x over previous
import functools
import jax
import jax.numpy as jnp
from jax import lax
from jax.experimental import pallas as pl
from jax.experimental.pallas import tpu as pltpu
from jax.experimental.pallas import tpu_sc as plsc

N, E, D, DE, H, L = 10000, 320000, 128, 16, 64, 3

_INTERPRET = False


def _node_body(x_ref, w1_ref, b1_ref, w2_ref, b2_ref, lng_ref, lnb_ref,
               encw_ref, encb_ref, bng_ref, bnb_ref, h_ref):
    t = jnp.maximum(x_ref[...] @ w1_ref[...] + b1_ref[...], 0.0)
    xp = t @ w2_ref[...] + b2_ref[...]
    m = jnp.mean(xp, axis=1, keepdims=True)
    v = jnp.mean((xp - m) ** 2, axis=1, keepdims=True)
    xp = (xp - m) / jnp.sqrt(v + 1e-5) * lng_ref[...] + lnb_ref[...]
    y = xp @ encw_ref[...] + encb_ref[...]
    mu = jnp.mean(y, axis=0, keepdims=True)
    var = jnp.mean((y - mu) ** 2, axis=0, keepdims=True)
    h_ref[...] = jnp.maximum(
        (y - mu) / jnp.sqrt(var + 1e-5) * bng_ref[...] + bnb_ref[...], 0.0)


def _node_dense(x, w1, b1, w2, b2, lng, lnb, encw, encb, bng, bnb):
    return pl.pallas_call(
        _node_body,
        out_shape=jax.ShapeDtypeStruct((N, H), jnp.float32),
        interpret=_INTERPRET,
    )(x, w1, b1.reshape(1, -1), w2, b2.reshape(1, -1), lng.reshape(1, -1),
      lnb.reshape(1, -1), encw, encb.reshape(1, -1), bng.reshape(1, -1),
      bnb.reshape(1, -1))


_BE = 6400


def _edge_body(eattr_ref, w1_ref, b1_ref, w2_ref, b2_ref, lng_ref, lnb_ref,
               lew_ref, ate_ref, ea32_ref, ae_ref, me_ref):
    t = jnp.maximum(eattr_ref[...] @ w1_ref[...] + b1_ref[...], 0.0)
    ea = t @ w2_ref[...] + b2_ref[...]
    m = jnp.mean(ea, axis=1, keepdims=True)
    v = jnp.mean((ea - m) ** 2, axis=1, keepdims=True)
    ea = (ea - m) / jnp.sqrt(v + 1e-5) * lng_ref[...] + lnb_ref[...]
    ea32_ref[...] = jnp.concatenate(
        [ea, jnp.full((ea.shape[0], 1), 1.0, jnp.float32),
         jnp.zeros((ea.shape[0], 32 - DE - 1), jnp.float32)], axis=1)
    lew = lew_ref[...]
    ate = ate_ref[...]
    we = jnp.sum(lew * ate[:, None, :], axis=2)
    ae = ea @ we.T
    ae_ref[...] = ae.T
    bmax = jnp.max(ae, axis=0, keepdims=True)
    @pl.when(pl.program_id(0) == 0)
    def _():
        me_ref[...] = bmax
    @pl.when(pl.program_id(0) != 0)
    def _():
        me_ref[...] = jnp.maximum(me_ref[...], bmax)


def _edge_dense(edge_attr, w1, b1, w2, b2, lng, lnb, lew, ate):
    nb = E // _BE
    return pl.pallas_call(
        _edge_body,
        grid=(nb,),
        in_specs=[
            pl.BlockSpec((_BE, DE), lambda i: (i, 0)),
            pl.BlockSpec((DE, 2 * DE), lambda i: (0, 0)),
            pl.BlockSpec((1, 2 * DE), lambda i: (0, 0)),
            pl.BlockSpec((2 * DE, DE), lambda i: (0, 0)),
            pl.BlockSpec((1, DE), lambda i: (0, 0)),
            pl.BlockSpec((1, DE), lambda i: (0, 0)),
            pl.BlockSpec((1, DE), lambda i: (0, 0)),
            pl.BlockSpec((L, DE, H), lambda i: (0, 0, 0)),
            pl.BlockSpec((L, H), lambda i: (0, 0)),
        ],
        out_specs=[
            pl.BlockSpec((_BE, 32), lambda i: (i, 0)),
            pl.BlockSpec((L, _BE), lambda i: (0, i)),
            pl.BlockSpec((1, L), lambda i: (0, 0)),
        ],
        out_shape=[
            jax.ShapeDtypeStruct((E, 32), jnp.float32),
            jax.ShapeDtypeStruct((L, E), jnp.float32),
            jax.ShapeDtypeStruct((1, L), jnp.float32),
        ],
        interpret=_INTERPRET,
    )(edge_attr, w1, b1.reshape(1, -1), w2, b2.reshape(1, -1),
      lng.reshape(1, -1), lnb.reshape(1, -1), lew, ate)


def _front_body(h_ref, linw_ref, ats_ref, atd_ref, me_ref,
                hw_ref, as_ref, ad_ref, m_ref):
    hw = h_ref[...] @ linw_ref[...]
    hw_ref[...] = jnp.concatenate(
        [hw, jnp.full((hw.shape[0], 1), 1.0, jnp.float32),
         jnp.zeros((hw.shape[0], 80 - H - 1), jnp.float32)], axis=1)
    a_s = hw @ ats_ref[...]
    a_d = hw @ atd_ref[...]
    as_ref[...] = a_s
    ad_ref[...] = a_d
    m = jnp.maximum(jnp.max(a_s) + jnp.max(a_d)
                    + jnp.maximum(me_ref[0, 0], 0.0), 0.0)
    m_ref[...] = jnp.full((1, 128), m, jnp.float32)


def _layer_front(h, linw, ats, atd, me_l):
    return pl.pallas_call(
        _front_body,
        out_shape=[
            jax.ShapeDtypeStruct((N, 80), jnp.float32),
            jax.ShapeDtypeStruct((N, 1), jnp.float32),
            jax.ShapeDtypeStruct((N, 1), jnp.float32),
            jax.ShapeDtypeStruct((1, 128), jnp.float32),
        ],
        interpret=_INTERPRET,
    )(h, linw, ats.reshape(-1, 1), atd.reshape(-1, 1), me_l.reshape(1, 1))


def _epilogue(u2, a_s, a_d, Sp, we_l, hw80, h_in, m, bias, g, b):
    def body(u_ref, as_ref, ad_ref, s_ref, we_ref, hw_ref, hin_ref,
             m_ref, bias_ref, g_ref, b_ref, h_ref):
        up = jnp.sum(u_ref[...], axis=0)
        u = up[:, :H]
        denom = up[:, H:H + 1]
        sp = jnp.sum(s_ref[...], axis=0)
        deg = sp[:, DE:DE + 1]
        ael = (sp[:, :DE] @ we_ref[...]) / jnp.maximum(deg, 1.0)
        zs = as_ref[...] + ad_ref[...] + ael
        zs = jnp.maximum(zs, 0.2 * zs)
        exs = jnp.exp(zs - m_ref[0, 0])
        hw = hw_ref[...][:, :H]
        out = (u + exs * hw) / (denom + exs + 1e-16) + bias_ref[...]
        mu = jnp.mean(out, axis=0, keepdims=True)
        var = jnp.mean((out - mu) ** 2, axis=0, keepdims=True)
        hn = jnp.maximum((out - mu) / jnp.sqrt(var + 1e-5) * g_ref[...]
                         + b_ref[...], 0.0)
        h_ref[...] = hin_ref[...] + hn
    return pl.pallas_call(
        body,
        out_shape=jax.ShapeDtypeStruct((N, H), jnp.float32),
        interpret=_INTERPRET,
    )(u2, a_s, a_d, Sp, we_l.reshape(-1, 1), hw80, h_in,
      m.reshape(1, -1), bias.reshape(1, -1), g.reshape(1, -1),
      b.reshape(1, -1))


def _final_body(h_ref, ow_ref, ob_ref, o_ref):
    g = jnp.mean(h_ref[...], axis=0, keepdims=True)
    o_ref[...] = jnp.maximum(g @ ow_ref[...] + ob_ref[...], 0.0)


def _final(h, out_w, out_b):
    return pl.pallas_call(
        _final_body,
        out_shape=jax.ShapeDtypeStruct((1, H), jnp.float32),
        interpret=_INTERPRET,
    )(h, out_w, out_b.reshape(1, -1))


_NC, _NS = 2, 16
_NW = _NC * _NS
_EW = E // _NW
_C = 128
_NCH = 80
_EP = _NCH * _C
_RPT = N // _NS
_SC_MESH = plsc.VectorSubcoreMesh(core_axis_name="c", subcore_axis_name="s")


def _zero_rows(ref, lo, nrows, width16):
    def body(i, _):
        for j in range(width16):
            ref[i, pl.ds(j * 16, 16)] = jnp.zeros((16,), jnp.float32)
        return 0
    lax.fori_loop(lo, lo + nrows, body, 0)


def _acc_blocks(sid):
    base = sid * _RPT
    return [(base, _C), (base + _C, _C), (base + 2 * _C, _C),
            (base + 3 * _C, _C), (base + 4 * _C, _RPT - 4 * _C)]


def _seg_body(dst3_hbm, ea32_hbm, sp_hbm, dstv, bufA, bufB, semA, semB,
              wsemA, wsemB, s_sh):
    cid = lax.axis_index("c")
    sid = lax.axis_index("s")
    wid = sid * _NC + cid
    bufs, sems, wsems = (bufA, bufB), (semA, semB), (wsemA, wsemB)
    pltpu.sync_copy(dst3_hbm.at[wid], dstv)
    _zero_rows(bufA, 0, _C, 2)
    for off, nr in _acc_blocks(sid):
        pltpu.sync_copy(bufA.at[pl.ds(0, nr)], s_sh.at[pl.ds(off, nr)])
    plsc.subcore_barrier()
    pltpu.async_copy(ea32_hbm.at[wid, 0], bufA, semA)

    def pair(g2, _):
        for b in (0, 1):
            ob = 1 - b
            c = g2 * 2 + b

            @pl.when(c >= 1)
            def _():
                pltpu.make_async_copy(
                    bufs[ob], s_sh.at[dstv.at[c - 1]], wsems[ob]).wait()

            @pl.when(c + 1 < _NCH)
            def _():
                pltpu.async_copy(ea32_hbm.at[wid, c + 1], bufs[ob], sems[ob])
            pltpu.make_async_copy(ea32_hbm.at[wid, c], bufs[b],
                                  sems[b]).wait()
            pltpu.async_copy(bufs[b], s_sh.at[dstv.at[c]], wsems[b],
                             add=True)
        return 0
    lax.fori_loop(0, _NCH // 2, pair, 0)
    pltpu.make_async_copy(bufs[1], s_sh.at[dstv.at[_NCH - 1]],
                          wsems[1]).wait()
    plsc.subcore_barrier()
    for off, nr in _acc_blocks(sid):
        pltpu.sync_copy(s_sh.at[pl.ds(off, nr)],
                        sp_hbm.at[cid, pl.ds(off, nr)])


def _seg_ea(dst3, ea32p):
    return pl.kernel(
        _seg_body,
        out_type=jax.ShapeDtypeStruct((_NC, N, 32), jnp.float32),
        mesh=_SC_MESH,
        compiler_params=pltpu.CompilerParams(use_tc_tiling_on_sc=False,
                                             needs_layout_passes=False),
        scratch_types=[
            pltpu.VMEM((_NCH, _C), jnp.int32),
            pltpu.VMEM((_C, 32), jnp.float32),
            pltpu.VMEM((_C, 32), jnp.float32),
            pltpu.SemaphoreType.DMA,
            pltpu.SemaphoreType.DMA,
            pltpu.SemaphoreType.DMA,
            pltpu.SemaphoreType.DMA,
            pltpu.VMEM_SHARED((N, 32), jnp.float32),
        ],
    )(dst3, ea32p)


def _gat_body(src3_hbm, dst3_hbm, ae3_hbm, as_hbm, ad_hbm, m_hbm, hw_hbm,
              up_hbm, asv, adv, srcv, dstv, aev, m16, exA, exB,
              rowsA, rowsB, gsemA, gsemB, ssemA, ssemB, u_sh):
    cid = lax.axis_index("c")
    sid = lax.axis_index("s")
    wid = sid * _NC + cid
    rows, exs = (rowsA, rowsB), (exA, exB)
    gsems, ssems = (gsemA, gsemB), (ssemA, ssemB)
    pltpu.sync_copy(as_hbm, asv)
    pltpu.sync_copy(ad_hbm, adv)
    pltpu.sync_copy(m_hbm.at[0, pl.ds(0, 16)], m16)
    pltpu.sync_copy(src3_hbm.at[wid], srcv)
    pltpu.sync_copy(dst3_hbm.at[wid], dstv)
    pltpu.sync_copy(ae3_hbm.at[wid], aev)
    _zero_rows(rowsA, 0, _C, 5)
    for off, nr in _acc_blocks(sid):
        pltpu.sync_copy(rowsA.at[pl.ds(0, nr)], u_sh.at[pl.ds(off, nr)])
    plsc.subcore_barrier()
    mvec = m16[...]
    pltpu.async_copy(hw_hbm.at[srcv.at[0]], rowsA, gsemA)

    def pair(g2, _):
        for b in (0, 1):
            ob = 1 - b
            c = g2 * 2 + b
            for i in range(8):
                sl = pl.ds(i * 16, 16)
                z = (plsc.load_gather(asv, [srcv[c, sl]])
                     + plsc.load_gather(adv, [dstv[c, sl]]) + aev[c, sl])
                z = jnp.maximum(z, 0.2 * z)
                exs[b][sl] = jnp.exp(z - mvec)

            @pl.when(c >= 1)
            def _():
                pltpu.make_async_copy(
                    rows[ob], u_sh.at[dstv.at[c - 1]], ssems[ob]).wait()

            @pl.when(c + 1 < _NCH)
            def _():
                pltpu.async_copy(hw_hbm.at[srcv.at[c + 1]], rows[ob],
                                 gsems[ob])
            pltpu.make_async_copy(hw_hbm.at[srcv.at[c]], rows[b],
                                  gsems[b]).wait()

            def scale4(e4, _):
                for k in range(4):
                    e = e4 * 4 + k
                    bc = plsc.load_gather(
                        exs[b], [jnp.full((16,), e, jnp.int32)])
                    for j in range(5):
                        sl = pl.ds(j * 16, 16)
                        rows[b][e, sl] = rows[b][e, sl] * bc
                return 0
            lax.fori_loop(0, _C // 4, scale4, 0)
            pltpu.async_copy(rows[b], u_sh.at[dstv.at[c]], ssems[b],
                             add=True)
        return 0
    lax.fori_loop(0, _NCH // 2, pair, 0)
    pltpu.make_async_copy(rows[1], u_sh.at[dstv.at[_NCH - 1]],
                          ssems[1]).wait()
    plsc.subcore_barrier()
    for off, nr in _acc_blocks(sid):
        pltpu.sync_copy(u_sh.at[pl.ds(off, nr)],
                        up_hbm.at[cid, pl.ds(off, nr)])


def _gat_edges(src3, dst3, ae3, a_s, a_d, m, hw80):
    return pl.kernel(
        _gat_body,
        out_type=jax.ShapeDtypeStruct((_NC, N, 80), jnp.float32),
        mesh=_SC_MESH,
        compiler_params=pltpu.CompilerParams(use_tc_tiling_on_sc=False,
                                             needs_layout_passes=False),
        scratch_types=[
            pltpu.VMEM((N,), jnp.float32),
            pltpu.VMEM((N,), jnp.float32),
            pltpu.VMEM((_NCH, _C), jnp.int32),
            pltpu.VMEM((_NCH, _C), jnp.int32),
            pltpu.VMEM((_NCH, _C), jnp.float32),
            pltpu.VMEM((16,), jnp.float32),
            pltpu.VMEM((_C,), jnp.float32),
            pltpu.VMEM((_C,), jnp.float32),
            pltpu.VMEM((_C, 80), jnp.float32),
            pltpu.VMEM((_C, 80), jnp.float32),
            pltpu.SemaphoreType.DMA,
            pltpu.SemaphoreType.DMA,
            pltpu.SemaphoreType.DMA,
            pltpu.SemaphoreType.DMA,
            pltpu.VMEM_SHARED((N, 80), jnp.float32),
        ],
    )(src3, dst3, ae3, a_s, a_d, m, hw80)


def kernel(x, edge_index, edge_attr, npp_w1, npp_b1, npp_w2, npp_b2,
           npp_ln_g, npp_ln_b, epp_w1, epp_b1, epp_w2, epp_b2, epp_ln_g,
           epp_ln_b, enc_w, enc_b, enc_bn_g, enc_bn_b, gat_lin_w,
           gat_att_src, gat_att_dst, gat_lin_edge_w, gat_att_edge,
           gat_bias, bn_g, bn_b, out_w, out_b):
    src, dst = edge_index[0], edge_index[1]
    h = _node_dense(x, npp_w1, npp_b1, npp_w2, npp_b2, npp_ln_g, npp_ln_b,
                    enc_w, enc_b, enc_bn_g, enc_bn_b)
    ea32, aeT, me = _edge_dense(edge_attr, epp_w1, epp_b1, epp_w2, epp_b2,
                                epp_ln_g, epp_ln_b, gat_lin_edge_w,
                                gat_att_edge)
    We = jnp.sum(gat_lin_edge_w * gat_att_edge[:, None, :], axis=2)
    pad = _EP - _EW
    src3 = jnp.concatenate([src.reshape(_NW, _EW),
                            jnp.zeros((_NW, pad), jnp.int32)],
                           axis=1).reshape(_NW, _NCH, _C)
    dst3 = jnp.concatenate([dst.reshape(_NW, _EW),
                            jnp.zeros((_NW, pad), jnp.int32)],
                           axis=1).reshape(_NW, _NCH, _C)
    ae3 = jnp.concatenate(
        [aeT.reshape(L, _NW, _EW),
         jnp.full((L, _NW, pad), -1e30, jnp.float32)],
        axis=2).reshape(L, _NW, _NCH, _C)
    ea32p = jnp.concatenate(
        [ea32.reshape(_NW, _EW, 32),
         jnp.zeros((_NW, pad, 32), jnp.float32)],
        axis=1).reshape(_NW, _NCH, _C, 32)
    Sp = _seg_ea(dst3, ea32p)

    for l in range(L):
        hw80, a_s, a_d, m = _layer_front(h, gat_lin_w[l], gat_att_src[l],
                                         gat_att_dst[l], me[0, l])
        u2 = _gat_edges(src3, dst3, ae3[l], a_s.reshape(N), a_d.reshape(N),
                        m, hw80)
        h = _epilogue(u2, a_s, a_d, Sp, We[l], hw80, h, m,
                      gat_bias[l], bn_g[l], bn_b[l])
    return _final(h, out_w, out_b)
